# Initial kernel scaffold; baseline (speedup 1.0000x reference)
import jax
import jax.numpy as jnp
from jax.experimental import pallas as pl


def kernel(x, pe):
    raise NotImplementedError("write your pallas kernel here")


# speedup vs baseline: 2.3853x; 2.3853x over previous
import jax
import jax.numpy as jnp
from jax.experimental import pallas as pl


_L_BLK = 256


def _add_body(x_ref, pe_ref, o_ref):
    o_ref[0] = x_ref[0] + pe_ref[...]


def kernel(x, pe):
    N, L, D = x.shape
    n_lblk = L // _L_BLK
    return pl.pallas_call(
        _add_body,
        grid=(n_lblk, N),
        in_specs=[
            pl.BlockSpec((1, _L_BLK, D), lambda l, n: (n, l, 0)),
            pl.BlockSpec((_L_BLK, D), lambda l, n: (l, 0)),
        ],
        out_specs=pl.BlockSpec((1, _L_BLK, D), lambda l, n: (n, l, 0)),
        out_shape=jax.ShapeDtypeStruct((N, L, D), x.dtype),
    )(x, pe)
